# Initial kernel scaffold; baseline (speedup 1.0000x reference)
import jax
import jax.numpy as jnp
from jax.experimental import pallas as pl


def kernel(boxes, scores):
    raise NotImplementedError("write your pallas kernel here")


# speedup vs baseline: 105.3634x; 105.3634x over previous
import functools

import jax
import jax.numpy as jnp
from jax import lax
from jax.experimental import pallas as pl

BLK = 128
THR = 0.5


def _iou_block(x1r, y1r, x2r, y2r, area_r, x1c, y1c, x2c, y2c, area_c):
    xx1 = jnp.maximum(x1r, x1c)
    yy1 = jnp.maximum(y1r, y1c)
    xx2 = jnp.minimum(x2r, x2c)
    yy2 = jnp.minimum(y2r, y2c)
    w = jnp.maximum(0.0, xx2 - xx1 + 1.0)
    h = jnp.maximum(0.0, yy2 - yy1 + 1.0)
    inter = w * h
    return inter / (area_r + area_c - inter)


def _nms_kernel(bs_ref, bt_ref, keep_ref):
    t = pl.program_id(0)

    @pl.when(t == 0)
    def _init():
        keep_ref[...] = jnp.zeros_like(keep_ref)

    base = pl.multiple_of(t * BLK, BLK)
    rows = bs_ref[pl.ds(base, BLK), :]
    x1r, y1r = rows[:, 0:1], rows[:, 1:2]
    x2r, y2r = rows[:, 2:3], rows[:, 3:4]
    area_r = (x2r - x1r + 1.0) * (y2r - y1r + 1.0)

    x1c, y1c = bt_ref[0:1, :], bt_ref[1:2, :]
    x2c, y2c = bt_ref[2:3, :], bt_ref[3:4, :]
    area_c = (x2c - x1c + 1.0) * (y2c - y1c + 1.0)

    iou = _iou_block(x1r, y1r, x2r, y2r, area_r, x1c, y1c, x2c, y2c, area_c)
    m = (iou > THR).astype(jnp.float32)

    keep_all = keep_ref[0:1, :]
    supp = jnp.sum(m * keep_all, axis=1, keepdims=True)
    alive = (supp == 0.0).astype(jnp.float32)

    x1l = bt_ref[0:1, pl.ds(base, BLK)]
    y1l = bt_ref[1:2, pl.ds(base, BLK)]
    x2l = bt_ref[2:3, pl.ds(base, BLK)]
    y2l = bt_ref[3:4, pl.ds(base, BLK)]
    area_l = (x2l - x1l + 1.0) * (y2l - y1l + 1.0)
    iou_l = _iou_block(x1r, y1r, x2r, y2r, area_r, x1l, y1l, x2l, y2l, area_l)
    lane = lax.broadcasted_iota(jnp.int32, (BLK, BLK), 1)
    subl = lax.broadcasted_iota(jnp.int32, (BLK, BLK), 0)
    m_strict = ((iou_l > THR) & (lane < subl)).astype(jnp.bfloat16)

    def cond(c):
        return c[0]

    def body(c):
        _, k = c
        sup = jnp.dot(m_strict, k.astype(jnp.bfloat16),
                      preferred_element_type=jnp.float32)
        new = alive * (sup == 0.0).astype(jnp.float32)
        return jnp.any(new != k), new

    _, keep_col = lax.while_loop(cond, body, (True, alive))

    eye = (lane == subl).astype(jnp.bfloat16)
    keep_row = lax.dot_general(
        keep_col.astype(jnp.bfloat16), eye,
        dimension_numbers=(((0,), (0,)), ((), ())),
        preferred_element_type=jnp.float32)
    keep_ref[0:1, pl.ds(base, BLK)] = keep_row


@functools.partial(jax.jit, static_argnames=("interpret",))
def _nms_keep(boxes_sorted_padded, interpret=False):
    p = boxes_sorted_padded.shape[0]
    grid = p // BLK
    bt = boxes_sorted_padded.T
    keep = pl.pallas_call(
        _nms_kernel,
        grid=(grid,),
        in_specs=[
            pl.BlockSpec((p, 4), lambda t: (0, 0)),
            pl.BlockSpec((4, p), lambda t: (0, 0)),
        ],
        out_specs=pl.BlockSpec((1, p), lambda t: (0, 0)),
        out_shape=jax.ShapeDtypeStruct((1, p), jnp.float32),
        interpret=interpret,
    )(boxes_sorted_padded, bt)
    return keep[0]


def kernel(boxes, scores, interpret=False):
    n = boxes.shape[0]
    p = ((n + BLK - 1) // BLK) * BLK
    order = jnp.argsort(-scores)
    bs = boxes[order]
    pad_box = jnp.array([-1e6, -1e6, -1e6 + 1.0, -1e6 + 1.0], jnp.float32)
    bs_p = jnp.concatenate(
        [bs, jnp.broadcast_to(pad_box, (p - n, 4))], axis=0)
    keep_sorted = _nms_keep(bs_p, interpret=interpret)[:n]
    keep = jnp.zeros((n,), jnp.float32).at[order].set(keep_sorted)
    out = jnp.concatenate(
        [boxes * keep[:, None], (scores * keep)[:, None]], axis=1)
    return out


# speedup vs baseline: 108.6896x; 1.0316x over previous
import functools

import jax
import jax.numpy as jnp
from jax import lax
from jax.experimental import pallas as pl

BLK = 128
CHUNK = 512
THR = 0.5


def _iou_block(x1r, y1r, x2r, y2r, area_r, x1c, y1c, x2c, y2c, area_c):
    xx1 = jnp.maximum(x1r, x1c)
    yy1 = jnp.maximum(y1r, y1c)
    xx2 = jnp.minimum(x2r, x2c)
    yy2 = jnp.minimum(y2r, y2c)
    w = jnp.maximum(0.0, xx2 - xx1 + 1.0)
    h = jnp.maximum(0.0, yy2 - yy1 + 1.0)
    inter = w * h
    return inter / (area_r + area_c - inter)


def _nms_kernel(bs_ref, bt_ref, keep_ref):
    t = pl.program_id(0)

    @pl.when(t == 0)
    def _init():
        keep_ref[...] = jnp.zeros_like(keep_ref)

    base = pl.multiple_of(t * BLK, BLK)
    rows = bs_ref[pl.ds(base, BLK), :]
    x1r, y1r = rows[:, 0:1], rows[:, 1:2]
    x2r, y2r = rows[:, 2:3], rows[:, 3:4]
    area_r = (x2r - x1r + 1.0) * (y2r - y1r + 1.0)

    def chunk_body(jb, acc):
        cb = pl.multiple_of(jb * CHUNK, CHUNK)
        x1c, y1c = bt_ref[0:1, pl.ds(cb, CHUNK)], bt_ref[1:2, pl.ds(cb, CHUNK)]
        x2c, y2c = bt_ref[2:3, pl.ds(cb, CHUNK)], bt_ref[3:4, pl.ds(cb, CHUNK)]
        area_c = bt_ref[4:5, pl.ds(cb, CHUNK)]
        iou = _iou_block(x1r, y1r, x2r, y2r, area_r,
                         x1c, y1c, x2c, y2c, area_c)
        m = (iou > THR).astype(jnp.float32)
        kc = keep_ref[0:1, pl.ds(cb, CHUNK)]
        return acc + jnp.sum(m * kc, axis=1, keepdims=True)

    nch = (t * BLK) // CHUNK + 1
    supp = lax.fori_loop(0, nch, chunk_body,
                         jnp.zeros((BLK, 1), jnp.float32))
    alive = (supp == 0.0).astype(jnp.float32)

    x1l = bt_ref[0:1, pl.ds(base, BLK)]
    y1l = bt_ref[1:2, pl.ds(base, BLK)]
    x2l = bt_ref[2:3, pl.ds(base, BLK)]
    y2l = bt_ref[3:4, pl.ds(base, BLK)]
    area_l = bt_ref[4:5, pl.ds(base, BLK)]
    iou_l = _iou_block(x1r, y1r, x2r, y2r, area_r, x1l, y1l, x2l, y2l, area_l)
    lane = lax.broadcasted_iota(jnp.int32, (BLK, BLK), 1)
    subl = lax.broadcasted_iota(jnp.int32, (BLK, BLK), 0)
    m_strict = ((iou_l > THR) & (lane < subl)).astype(jnp.bfloat16)

    def cond(c):
        return c[0]

    def body(c):
        _, k = c
        sup = jnp.dot(m_strict, k.astype(jnp.bfloat16),
                      preferred_element_type=jnp.float32)
        new = alive * (sup == 0.0).astype(jnp.float32)
        return jnp.any(new != k), new

    _, keep_col = lax.while_loop(cond, body, (True, alive))

    eye = (lane == subl).astype(jnp.bfloat16)
    keep_row = lax.dot_general(
        keep_col.astype(jnp.bfloat16), eye,
        dimension_numbers=(((0,), (0,)), ((), ())),
        preferred_element_type=jnp.float32)
    keep_ref[0:1, pl.ds(base, BLK)] = keep_row


@functools.partial(jax.jit, static_argnames=("interpret",))
def _nms_keep(boxes_sorted_padded, interpret=False):
    p = boxes_sorted_padded.shape[0]
    grid = p // BLK
    b = boxes_sorted_padded
    area = (b[:, 2] - b[:, 0] + 1.0) * (b[:, 3] - b[:, 1] + 1.0)
    bt = jnp.concatenate([b.T, area[None, :]], axis=0)
    keep = pl.pallas_call(
        _nms_kernel,
        grid=(grid,),
        in_specs=[
            pl.BlockSpec((p, 4), lambda t: (0, 0)),
            pl.BlockSpec((5, p), lambda t: (0, 0)),
        ],
        out_specs=pl.BlockSpec((1, p), lambda t: (0, 0)),
        out_shape=jax.ShapeDtypeStruct((1, p), jnp.float32),
        interpret=interpret,
    )(boxes_sorted_padded, bt)
    return keep[0]


def kernel(boxes, scores, interpret=False):
    n = boxes.shape[0]
    p = ((n + BLK - 1) // BLK) * BLK
    order = jnp.argsort(-scores)
    bs = boxes[order]
    pad_box = jnp.array([-1e6, -1e6, -1e6 + 1.0, -1e6 + 1.0], jnp.float32)
    bs_p = jnp.concatenate(
        [bs, jnp.broadcast_to(pad_box, (p - n, 4))], axis=0)
    keep_sorted = _nms_keep(bs_p, interpret=interpret)[:n]
    keep = jnp.zeros((n,), jnp.float32).at[order].set(keep_sorted)
    out = jnp.concatenate(
        [boxes * keep[:, None], (scores * keep)[:, None]], axis=1)
    return out


# speedup vs baseline: 139.9652x; 1.2878x over previous
import functools

import jax
import jax.numpy as jnp
from jax import lax
from jax.experimental import pallas as pl

BLK = 512
CHUNK = 512
THR = 0.5


def _iou_block(x1r, y1r, x2r, y2r, area_r, x1c, y1c, x2c, y2c, area_c):
    xx1 = jnp.maximum(x1r, x1c)
    yy1 = jnp.maximum(y1r, y1c)
    xx2 = jnp.minimum(x2r, x2c)
    yy2 = jnp.minimum(y2r, y2c)
    w = jnp.maximum(0.0, xx2 - xx1 + 1.0)
    h = jnp.maximum(0.0, yy2 - yy1 + 1.0)
    inter = w * h
    return inter / (area_r + area_c - inter)


def _nms_kernel(bs_ref, bt_ref, keep_ref):
    t = pl.program_id(0)

    @pl.when(t == 0)
    def _init():
        keep_ref[...] = jnp.zeros_like(keep_ref)

    base = pl.multiple_of(t * BLK, BLK)
    rows = bs_ref[pl.ds(base, BLK), :]
    x1r, y1r = rows[:, 0:1], rows[:, 1:2]
    x2r, y2r = rows[:, 2:3], rows[:, 3:4]
    area_r = (x2r - x1r + 1.0) * (y2r - y1r + 1.0)

    def chunk_body(jb, acc):
        cb = pl.multiple_of(jb * CHUNK, CHUNK)
        x1c, y1c = bt_ref[0:1, pl.ds(cb, CHUNK)], bt_ref[1:2, pl.ds(cb, CHUNK)]
        x2c, y2c = bt_ref[2:3, pl.ds(cb, CHUNK)], bt_ref[3:4, pl.ds(cb, CHUNK)]
        area_c = bt_ref[4:5, pl.ds(cb, CHUNK)]
        iou = _iou_block(x1r, y1r, x2r, y2r, area_r,
                         x1c, y1c, x2c, y2c, area_c)
        m = (iou > THR).astype(jnp.float32)
        kc = keep_ref[0:1, pl.ds(cb, CHUNK)]
        return acc + jnp.sum(m * kc, axis=1, keepdims=True)

    nch = (t * BLK) // CHUNK + 1
    supp = lax.fori_loop(0, nch, chunk_body,
                         jnp.zeros((BLK, 1), jnp.float32))
    alive = (supp == 0.0).astype(jnp.float32)

    x1l = bt_ref[0:1, pl.ds(base, BLK)]
    y1l = bt_ref[1:2, pl.ds(base, BLK)]
    x2l = bt_ref[2:3, pl.ds(base, BLK)]
    y2l = bt_ref[3:4, pl.ds(base, BLK)]
    area_l = bt_ref[4:5, pl.ds(base, BLK)]
    iou_l = _iou_block(x1r, y1r, x2r, y2r, area_r, x1l, y1l, x2l, y2l, area_l)
    lane = lax.broadcasted_iota(jnp.int32, (BLK, BLK), 1)
    subl = lax.broadcasted_iota(jnp.int32, (BLK, BLK), 0)
    m_strict = ((iou_l > THR) & (lane < subl)).astype(jnp.bfloat16)

    def cond(c):
        return c[0]

    def body(c):
        _, k = c
        sup = jnp.dot(m_strict, k.astype(jnp.bfloat16),
                      preferred_element_type=jnp.float32)
        new = alive * (sup == 0.0).astype(jnp.float32)
        return jnp.any(new != k), new

    _, keep_col = lax.while_loop(cond, body, (True, alive))

    eye = (lane == subl).astype(jnp.bfloat16)
    keep_row = lax.dot_general(
        keep_col.astype(jnp.bfloat16), eye,
        dimension_numbers=(((0,), (0,)), ((), ())),
        preferred_element_type=jnp.float32)
    keep_ref[0:1, pl.ds(base, BLK)] = keep_row


@functools.partial(jax.jit, static_argnames=("interpret",))
def _nms_keep(boxes_sorted_padded, interpret=False):
    p = boxes_sorted_padded.shape[0]
    grid = p // BLK
    b = boxes_sorted_padded
    area = (b[:, 2] - b[:, 0] + 1.0) * (b[:, 3] - b[:, 1] + 1.0)
    bt = jnp.concatenate([b.T, area[None, :]], axis=0)
    keep = pl.pallas_call(
        _nms_kernel,
        grid=(grid,),
        in_specs=[
            pl.BlockSpec((p, 4), lambda t: (0, 0)),
            pl.BlockSpec((5, p), lambda t: (0, 0)),
        ],
        out_specs=pl.BlockSpec((1, p), lambda t: (0, 0)),
        out_shape=jax.ShapeDtypeStruct((1, p), jnp.float32),
        interpret=interpret,
    )(boxes_sorted_padded, bt)
    return keep[0]


def kernel(boxes, scores, interpret=False):
    n = boxes.shape[0]
    p = ((n + BLK - 1) // BLK) * BLK
    order = jnp.argsort(-scores)
    bs = boxes[order]
    pad_box = jnp.array([-1e6, -1e6, -1e6 + 1.0, -1e6 + 1.0], jnp.float32)
    bs_p = jnp.concatenate(
        [bs, jnp.broadcast_to(pad_box, (p - n, 4))], axis=0)
    keep_sorted = _nms_keep(bs_p, interpret=interpret)[:n]
    keep = jnp.zeros((n,), jnp.float32).at[order].set(keep_sorted)
    out = jnp.concatenate(
        [boxes * keep[:, None], (scores * keep)[:, None]], axis=1)
    return out
